# Initial kernel scaffold; baseline (speedup 1.0000x reference)
import jax
import jax.numpy as jnp
from jax.experimental import pallas as pl


def kernel(X, L_u, L_d, W_s, W_u, W_d):
    raise NotImplementedError("write your pallas kernel here")


# speedup vs baseline: 1.0272x; 1.0272x over previous
import functools

import jax
import jax.numpy as jnp
from jax import lax
from jax.experimental import pallas as pl
from jax.experimental.pallas import tpu as pltpu

_N = 4096
_D = 64
_BJ = 256

_NT_DIMS = (((1,), (1,)), ((), ()))


def _phase_a_body(x_ref, ws_ref, wu_ref, wd_ref, hs_ref, yu_ref, yd_ref):
    x = x_ref[...]
    hs_ref[...] = jnp.tanh(
        lax.dot_general(x, ws_ref[...], _NT_DIMS, preferred_element_type=jnp.float32)
    )
    yu_ref[...] = lax.dot_general(
        x, wu_ref[...], _NT_DIMS, preferred_element_type=jnp.float32
    )
    yd_ref[...] = lax.dot_general(
        x, wd_ref[...], _NT_DIMS, preferred_element_type=jnp.float32
    )


def _phase_b_body(hs_ref, yu_ref, yd_ref, lu_ref, ld_ref, out_ref):
    zu = jnp.dot(yu_ref[...], lu_ref[...], preferred_element_type=jnp.float32)
    zd = jnp.dot(yd_ref[...], ld_ref[...], preferred_element_type=jnp.float32)
    out_ref[...] = hs_ref[...] + jnp.tanh(zu) + jnp.tanh(zd)


@functools.partial(jax.jit, static_argnames=())
def kernel(X, L_u, L_d, W_s, W_u, W_d):
    nblocks = _N // _BJ
    f32 = jnp.float32

    hs, yu, yd = pl.pallas_call(
        _phase_a_body,
        grid=(nblocks,),
        in_specs=[
            pl.BlockSpec((_D, _N), lambda j: (0, 0)),
            pl.BlockSpec((_BJ, _N), lambda j: (j, 0)),
            pl.BlockSpec((_BJ, _N), lambda j: (j, 0)),
            pl.BlockSpec((_BJ, _N), lambda j: (j, 0)),
        ],
        out_specs=[
            pl.BlockSpec((_D, _BJ), lambda j: (0, j)),
            pl.BlockSpec((_D, _BJ), lambda j: (0, j)),
            pl.BlockSpec((_D, _BJ), lambda j: (0, j)),
        ],
        out_shape=[
            jax.ShapeDtypeStruct((_D, _N), f32),
            jax.ShapeDtypeStruct((_D, _N), f32),
            jax.ShapeDtypeStruct((_D, _N), f32),
        ],
        compiler_params=pltpu.CompilerParams(
            dimension_semantics=("arbitrary",),
        ),
    )(X, W_s, W_u, W_d)

    out = pl.pallas_call(
        _phase_b_body,
        grid=(nblocks,),
        in_specs=[
            pl.BlockSpec((_D, _BJ), lambda j: (0, j)),
            pl.BlockSpec((_D, _N), lambda j: (0, 0)),
            pl.BlockSpec((_D, _N), lambda j: (0, 0)),
            pl.BlockSpec((_N, _BJ), lambda j: (0, j)),
            pl.BlockSpec((_N, _BJ), lambda j: (0, j)),
        ],
        out_specs=pl.BlockSpec((_D, _BJ), lambda j: (0, j)),
        out_shape=jax.ShapeDtypeStruct((_D, _N), f32),
        compiler_params=pltpu.CompilerParams(
            dimension_semantics=("arbitrary",),
        ),
    )(hs, yu, yd, L_u, L_d)
    return out


# speedup vs baseline: 1.0312x; 1.0039x over previous
import functools

import jax
import jax.numpy as jnp
from jax import lax
from jax.experimental import pallas as pl
from jax.experimental.pallas import tpu as pltpu

_N = 4096
_D = 64
_BJ = 256

_NT_DIMS = (((1,), (1,)), ((), ()))


def _phase_a_body(x_ref, ws_ref, wu_ref, wd_ref, hs_ref, yu_ref, yd_ref):
    x = x_ref[...].astype(jnp.bfloat16)
    hs_ref[...] = jnp.tanh(
        lax.dot_general(
            x,
            ws_ref[...].astype(jnp.bfloat16),
            _NT_DIMS,
            preferred_element_type=jnp.float32,
        )
    )
    yu_ref[...] = lax.dot_general(
        x,
        wu_ref[...].astype(jnp.bfloat16),
        _NT_DIMS,
        preferred_element_type=jnp.float32,
    )
    yd_ref[...] = lax.dot_general(
        x,
        wd_ref[...].astype(jnp.bfloat16),
        _NT_DIMS,
        preferred_element_type=jnp.float32,
    )


def _phase_b_body(hs_ref, yu_ref, yd_ref, lu_ref, ld_ref, out_ref):
    zu = jnp.dot(
        yu_ref[...].astype(jnp.bfloat16),
        lu_ref[...].astype(jnp.bfloat16),
        preferred_element_type=jnp.float32,
    )
    zd = jnp.dot(
        yd_ref[...].astype(jnp.bfloat16),
        ld_ref[...].astype(jnp.bfloat16),
        preferred_element_type=jnp.float32,
    )
    out_ref[...] = hs_ref[...] + jnp.tanh(zu) + jnp.tanh(zd)


@functools.partial(jax.jit, static_argnames=())
def kernel(X, L_u, L_d, W_s, W_u, W_d):
    nblocks = _N // _BJ
    f32 = jnp.float32

    hs, yu, yd = pl.pallas_call(
        _phase_a_body,
        grid=(nblocks,),
        in_specs=[
            pl.BlockSpec((_D, _N), lambda j: (0, 0)),
            pl.BlockSpec((_BJ, _N), lambda j: (j, 0)),
            pl.BlockSpec((_BJ, _N), lambda j: (j, 0)),
            pl.BlockSpec((_BJ, _N), lambda j: (j, 0)),
        ],
        out_specs=[
            pl.BlockSpec((_D, _BJ), lambda j: (0, j)),
            pl.BlockSpec((_D, _BJ), lambda j: (0, j)),
            pl.BlockSpec((_D, _BJ), lambda j: (0, j)),
        ],
        out_shape=[
            jax.ShapeDtypeStruct((_D, _N), f32),
            jax.ShapeDtypeStruct((_D, _N), f32),
            jax.ShapeDtypeStruct((_D, _N), f32),
        ],
        compiler_params=pltpu.CompilerParams(
            dimension_semantics=("arbitrary",),
        ),
    )(X, W_s, W_u, W_d)

    out = pl.pallas_call(
        _phase_b_body,
        grid=(nblocks,),
        in_specs=[
            pl.BlockSpec((_D, _BJ), lambda j: (0, j)),
            pl.BlockSpec((_D, _N), lambda j: (0, 0)),
            pl.BlockSpec((_D, _N), lambda j: (0, 0)),
            pl.BlockSpec((_N, _BJ), lambda j: (0, j)),
            pl.BlockSpec((_N, _BJ), lambda j: (0, j)),
        ],
        out_specs=pl.BlockSpec((_D, _BJ), lambda j: (0, j)),
        out_shape=jax.ShapeDtypeStruct((_D, _N), f32),
        compiler_params=pltpu.CompilerParams(
            dimension_semantics=("arbitrary",),
        ),
    )(hs, yu, yd, L_u, L_d)
    return out


# speedup vs baseline: 1.0642x; 1.0320x over previous
import functools

import jax
import jax.numpy as jnp
from jax import lax
from jax.experimental import pallas as pl
from jax.experimental.pallas import tpu as pltpu

_N = 4096
_D = 64
_BJ = 256
_NB = _N // _BJ

_NT_DIMS = (((1,), (1,)), ((), ()))


def _body(x_ref, ws_ref, wu_ref, wd_ref, lu_ref, ld_ref, out_ref,
          hs_ref, yu_ref, yd_ref):
    phase = pl.program_id(0)
    j = pl.program_id(1)

    @pl.when(phase == 0)
    def _():
        x = x_ref[...].astype(jnp.bfloat16)
        cols = pl.ds(j * _BJ, _BJ)
        hs_ref[:, cols] = jnp.tanh(
            lax.dot_general(x, ws_ref[...].astype(jnp.bfloat16), _NT_DIMS,
                            preferred_element_type=jnp.float32))
        yu_ref[:, cols] = lax.dot_general(
            x, wu_ref[...].astype(jnp.bfloat16), _NT_DIMS,
            preferred_element_type=jnp.float32).astype(jnp.bfloat16)
        yd_ref[:, cols] = lax.dot_general(
            x, wd_ref[...].astype(jnp.bfloat16), _NT_DIMS,
            preferred_element_type=jnp.float32).astype(jnp.bfloat16)

    @pl.when(phase == 1)
    def _():
        zu = jnp.dot(yu_ref[...], lu_ref[...].astype(jnp.bfloat16),
                     preferred_element_type=jnp.float32)
        zd = jnp.dot(yd_ref[...], ld_ref[...].astype(jnp.bfloat16),
                     preferred_element_type=jnp.float32)
        out_ref[...] = hs_ref[:, pl.ds(j * _BJ, _BJ)] + jnp.tanh(zu) + jnp.tanh(zd)


@functools.partial(jax.jit, static_argnames=())
def kernel(X, L_u, L_d, W_s, W_u, W_d):
    f32 = jnp.float32

    def w_idx(p, j):
        return (jnp.where(p == 0, j, _NB - 1), 0)

    def l_idx(p, j):
        return (0, jnp.where(p == 0, 0, j))

    return pl.pallas_call(
        _body,
        grid=(2, _NB),
        in_specs=[
            pl.BlockSpec((_D, _N), lambda p, j: (0, 0)),
            pl.BlockSpec((_BJ, _N), w_idx),
            pl.BlockSpec((_BJ, _N), w_idx),
            pl.BlockSpec((_BJ, _N), w_idx),
            pl.BlockSpec((_N, _BJ), l_idx),
            pl.BlockSpec((_N, _BJ), l_idx),
        ],
        out_specs=pl.BlockSpec((_D, _BJ), l_idx),
        out_shape=jax.ShapeDtypeStruct((_D, _N), f32),
        scratch_shapes=[
            pltpu.VMEM((_D, _N), f32),
            pltpu.VMEM((_D, _N), jnp.bfloat16),
            pltpu.VMEM((_D, _N), jnp.bfloat16),
        ],
        compiler_params=pltpu.CompilerParams(
            dimension_semantics=("arbitrary", "arbitrary"),
        ),
    )(X, W_s, W_u, W_d, L_u, L_d)
